# Initial kernel scaffold; baseline (speedup 1.0000x reference)
import functools

import jax
import jax.numpy as jnp
from jax import lax
from jax.experimental import pallas as pl
from jax.experimental.pallas import tpu as pltpu
from jax.experimental.pallas import tpu_sc as plsc

N = 10000
E = 320000
D = 128
TW = 80
OW = 144
CB = 128
NCHUNK = E // CB
NWORK = 32
RPT = N // 16
ZB = 125


def _silu(x):
    return x * jax.nn.sigmoid(x)


def _dot_t(x, w):
    return lax.dot_general(x, w, (((1,), (1,)), ((), ())),
                           preferred_element_type=jnp.float32)


def _tc_pre_body(h_ref, c_ref, a_ref, b_ref, ta_ref, tb_ref):
    h = h_ref[...]
    cd = c_ref[...]
    pad = jnp.zeros((h.shape[0], TW - 67), jnp.float32)
    ha = _dot_t(h, a_ref[...])
    hb = _dot_t(h, b_ref[...])
    ta_ref[...] = jnp.concatenate([ha, cd, pad], axis=1)
    tb_ref[...] = jnp.concatenate([hb, -cd, pad], axis=1)


def _tc_pre(h, coord, a, b):
    blk = 2000
    grid = N // blk
    return pl.pallas_call(
        _tc_pre_body,
        grid=(grid,),
        in_specs=[
            pl.BlockSpec((blk, D), lambda i: (i, 0)),
            pl.BlockSpec((blk, 3), lambda i: (i, 0)),
            pl.BlockSpec((64, D), lambda i: (0, 0)),
            pl.BlockSpec((64, D), lambda i: (0, 0)),
        ],
        out_specs=[
            pl.BlockSpec((blk, TW), lambda i: (i, 0)),
            pl.BlockSpec((blk, TW), lambda i: (i, 0)),
        ],
        out_shape=[
            jax.ShapeDtypeStruct((N, TW), jnp.float32),
            jax.ShapeDtypeStruct((N, TW), jnp.float32),
        ],
    )(h, coord, a, b)


def _sc_gather(ta, tb, row, col):
    mesh = plsc.VectorSubcoreMesh(core_axis_name="c", subcore_axis_name="s")

    @functools.partial(
        pl.kernel,
        out_type=[
            jax.ShapeDtypeStruct((E, TW), jnp.float32),
            jax.ShapeDtypeStruct((E, TW), jnp.float32),
        ],
        mesh=mesh,
        scratch_types=[
            pltpu.VMEM((CB,), jnp.int32),
            pltpu.VMEM((CB,), jnp.int32),
            pltpu.VMEM((CB, TW), jnp.float32),
            pltpu.VMEM((CB, TW), jnp.float32),
            pltpu.SemaphoreType.DMA,
            pltpu.SemaphoreType.DMA,
        ],
    )
    def k(ta_hbm, tb_hbm, row_hbm, col_hbm, ga_hbm, gb_hbm,
          idr, idc, bufa, bufb, sema, semb):
        cid = lax.axis_index("c")
        sid = lax.axis_index("s")
        wid = sid * 2 + cid

        @pl.loop(0, (NCHUNK + NWORK - 1) // NWORK)
        def _(t):
            ch = wid + t * NWORK

            @pl.when(ch < NCHUNK)
            def _():
                base = ch * CB
                pltpu.sync_copy(row_hbm.at[pl.ds(base, CB)], idr)
                pltpu.sync_copy(col_hbm.at[pl.ds(base, CB)], idc)
                ca = pltpu.async_copy(ta_hbm.at[idr], bufa, sema)
                cb = pltpu.async_copy(tb_hbm.at[idc], bufb, semb)
                ca.wait()
                cb.wait()
                pltpu.sync_copy(bufa, ga_hbm.at[pl.ds(base, CB)])
                pltpu.sync_copy(bufb, gb_hbm.at[pl.ds(base, CB)])

    return k(ta, tb, row, col)


def _tc_edge_body(ga_ref, gb_ref, ea_ref,
                  wr_ref, c4_ref, eb0_ref, ew1_ref, eb1_ref, ew2_ref, eb2_ref,
                  cw0_ref, cb0_ref, cw1_ref, cb1_ref, cw2_ref,
                  out_ref):
    s = ga_ref[...] + gb_ref[...]
    z0p = s[:, :64]
    cdiff = s[:, 64:67]
    radial = jnp.sum(cdiff * cdiff, axis=1, keepdims=True)
    ea = ea_ref[...]
    z0 = z0p + radial * wr_ref[...] + jnp.dot(
        ea, c4_ref[...], preferred_element_type=jnp.float32) + eb0_ref[...]
    z0 = _silu(z0)
    z1 = _silu(_dot_t(z0, ew1_ref[...]) + eb1_ref[...])
    ef = _silu(_dot_t(z1, ew2_ref[...]) + eb2_ref[...])
    c0 = _silu(_dot_t(ef, cw0_ref[...]) + cb0_ref[...])
    c1 = _silu(_dot_t(c0, cw1_ref[...]) + cb1_ref[...])
    cc = jnp.sum(c1 * cw2_ref[...], axis=1, keepdims=True)
    trans = jnp.clip(cdiff * cc, -100.0, 100.0)
    nb = s.shape[0]
    ones = jnp.ones((nb, 1), jnp.float32)
    zpad = jnp.zeros((nb, OW - 132), jnp.float32)
    out_ref[...] = jnp.concatenate([ef, trans, ones, zpad], axis=1)


def _tc_edge(ga, gb, ea, wr, c4, eb0, ew1, eb1, ew2, eb2,
             cw0, cb0, cw1, cb1, cw2):
    blk = 2560
    grid = E // blk
    full = lambda shp: pl.BlockSpec(shp, lambda i: tuple(0 for _ in shp))
    return pl.pallas_call(
        _tc_edge_body,
        grid=(grid,),
        in_specs=[
            pl.BlockSpec((blk, TW), lambda i: (i, 0)),
            pl.BlockSpec((blk, TW), lambda i: (i, 0)),
            pl.BlockSpec((blk, 4), lambda i: (i, 0)),
            full(wr.shape), full(c4.shape), full(eb0.shape),
            full(ew1.shape), full(eb1.shape), full(ew2.shape), full(eb2.shape),
            full(cw0.shape), full(cb0.shape), full(cw1.shape), full(cb1.shape),
            full(cw2.shape),
        ],
        out_specs=pl.BlockSpec((blk, OW), lambda i: (i, 0)),
        out_shape=jax.ShapeDtypeStruct((E, OW), jnp.float32),
    )(ga, gb, ea, wr, c4, eb0, ew1, eb1, ew2, eb2, cw0, cb0, cw1, cb1, cw2)


def _sc_scatter(out_e, row):
    mesh = plsc.VectorSubcoreMesh(core_axis_name="c", subcore_axis_name="s")
    halfc = NCHUNK // 2

    @functools.partial(
        pl.kernel,
        out_type=jax.ShapeDtypeStruct((2, N, OW), jnp.float32),
        mesh=mesh,
        scratch_types=[
            pltpu.VMEM((CB,), jnp.int32),
            pltpu.VMEM((CB, OW), jnp.float32),
            pltpu.VMEM((ZB, OW), jnp.float32),
            pltpu.VMEM_SHARED((N, OW), jnp.float32),
        ],
    )
    def k(oe_hbm, row_hbm, p_hbm, idx, buf, zbuf, acc):
        cid = lax.axis_index("c")
        sid = lax.axis_index("s")

        @pl.loop(0, ZB)
        def _(i):
            @pl.loop(0, OW // 16)
            def _(j):
                zbuf[i, pl.ds(j * 16, 16)] = jnp.zeros((16,), jnp.float32)

        @pl.loop(0, RPT // ZB)
        def _(kk):
            pltpu.sync_copy(zbuf, acc.at[pl.ds(sid * RPT + kk * ZB, ZB)])

        plsc.subcore_barrier()

        @pl.loop(0, (halfc + 15) // 16)
        def _(t):
            kchunk = sid + t * 16

            @pl.when(kchunk < halfc)
            def _():
                base = (cid * halfc + kchunk) * CB
                pltpu.sync_copy(row_hbm.at[pl.ds(base, CB)], idx)
                pltpu.sync_copy(oe_hbm.at[pl.ds(base, CB)], buf)
                pltpu.sync_copy(buf, acc.at[idx], add=True)

        plsc.subcore_barrier()

        @pl.loop(0, RPT // ZB)
        def _(kk):
            off = sid * RPT + kk * ZB
            pltpu.sync_copy(acc.at[pl.ds(off, ZB)], zbuf)
            pltpu.sync_copy(zbuf, p_hbm.at[cid, pl.ds(off, ZB)])

    return k(out_e, row)


def _tc_post_body(h_ref, c_ref, p_ref,
                  nw0_ref, nb0_ref, nw1_ref, nb1_ref, nw2_ref, nb2_ref,
                  hout_ref, cout_ref):
    acc = p_ref[0] + p_ref[1]
    agg = acc[:, :D]
    ts = acc[:, D:D + 3]
    cnt = acc[:, D + 3:D + 4]
    h = h_ref[...]
    nin = jnp.concatenate([h, agg], axis=1)
    y = _silu(_dot_t(nin, nw0_ref[...]) + nb0_ref[...])
    y = _silu(_dot_t(y, nw1_ref[...]) + nb1_ref[...])
    y = _dot_t(y, nw2_ref[...]) + nb2_ref[...]
    hout_ref[...] = h + y
    cout_ref[...] = c_ref[...] + ts / jnp.clip(cnt, 1.0, None)


def _tc_post(h, coord, p, nw0, nb0, nw1, nb1, nw2, nb2):
    blk = 2000
    grid = N // blk
    full = lambda shp: pl.BlockSpec(shp, lambda i: tuple(0 for _ in shp))
    return pl.pallas_call(
        _tc_post_body,
        grid=(grid,),
        in_specs=[
            pl.BlockSpec((blk, D), lambda i: (i, 0)),
            pl.BlockSpec((blk, 3), lambda i: (i, 0)),
            pl.BlockSpec((2, blk, OW), lambda i: (0, i, 0)),
            full(nw0.shape), full(nb0.shape), full(nw1.shape),
            full(nb1.shape), full(nw2.shape), full(nb2.shape),
        ],
        out_specs=[
            pl.BlockSpec((blk, D), lambda i: (i, 0)),
            pl.BlockSpec((blk, 3), lambda i: (i, 0)),
        ],
        out_shape=[
            jax.ShapeDtypeStruct((N, D), jnp.float32),
            jax.ShapeDtypeStruct((N, 3), jnp.float32),
        ],
    )(h, coord, p, nw0, nb0, nw1, nb1, nw2, nb2)


def kernel(h, edge_index, coord, edge_attr,
           ew0, eb0, ew1, eb1, ew2, eb2,
           nw0, nb0, nw1, nb1, nw2, nb2,
           cw0, cb0, cw1, cb1, cw2):
    row = edge_index[0]
    col = edge_index[1]
    a = ew0[:, :D]
    b = ew0[:, D:2 * D]
    wr = ew0[:, 2 * D:2 * D + 1].T
    c4 = ew0[:, 2 * D + 1:].T

    ta, tb = _tc_pre(h, coord, a, b)
    ga, gb = _sc_gather(ta, tb, row, col)
    out_e = _tc_edge(ga, gb, edge_attr, wr, c4,
                     eb0.reshape(1, -1), ew1, eb1.reshape(1, -1),
                     ew2, eb2.reshape(1, -1),
                     cw0, cb0.reshape(1, -1), cw1, cb1.reshape(1, -1), cw2)
    p = _sc_scatter(out_e, row)
    h_out, coord_out = _tc_post(h, coord, p, nw0, nb0.reshape(1, -1),
                                nw1, nb1.reshape(1, -1), nw2, nb2.reshape(1, -1))
    return (h_out, coord_out, edge_attr)


# speedup vs baseline: 4.5452x; 4.5452x over previous
import functools

import jax
import jax.numpy as jnp
from jax import lax
from jax.experimental import pallas as pl
from jax.experimental.pallas import tpu as pltpu
from jax.experimental.pallas import tpu_sc as plsc

N = 10000
E = 320000
D = 128
TW = 128
OW = 128
CB = 128
NCHUNK = E // CB
NWORK = 32
ZB = 200
NZB = N // ZB


def _silu(x):
    return x * jax.nn.sigmoid(x)


def _dot_t(x, w):
    return lax.dot_general(x, w, (((1,), (1,)), ((), ())),
                           preferred_element_type=jnp.float32)


def _tc_pre_body(h_ref, c_ref, a_ref, b_ref, ta_ref, tb_ref):
    h = h_ref[...]
    cd = c_ref[...]
    pad = jnp.zeros((h.shape[0], TW - 67), jnp.float32)
    ha = _dot_t(h, a_ref[...])
    hb = _dot_t(h, b_ref[...])
    ta_ref[...] = jnp.concatenate([ha, cd, pad], axis=1)
    tb_ref[...] = jnp.concatenate([hb, -cd, pad], axis=1)


def _tc_pre(h, coord, a, b):
    blk = 2000
    grid = N // blk
    return pl.pallas_call(
        _tc_pre_body,
        grid=(grid,),
        in_specs=[
            pl.BlockSpec((blk, D), lambda i: (i, 0)),
            pl.BlockSpec((blk, 3), lambda i: (i, 0)),
            pl.BlockSpec((64, D), lambda i: (0, 0)),
            pl.BlockSpec((64, D), lambda i: (0, 0)),
        ],
        out_specs=[
            pl.BlockSpec((blk, TW), lambda i: (i, 0)),
            pl.BlockSpec((blk, TW), lambda i: (i, 0)),
        ],
        out_shape=[
            jax.ShapeDtypeStruct((N, TW), jnp.float32),
            jax.ShapeDtypeStruct((N, TW), jnp.float32),
        ],
    )(h, coord, a, b)


def _sc_gather(ta, tb, row, col):
    mesh = plsc.VectorSubcoreMesh(core_axis_name="c", subcore_axis_name="s")

    @functools.partial(
        pl.kernel,
        out_type=[
            jax.ShapeDtypeStruct((E, TW), jnp.float32),
            jax.ShapeDtypeStruct((E, TW), jnp.float32),
        ],
        mesh=mesh,
        scratch_types=[
            pltpu.VMEM((CB,), jnp.int32),
            pltpu.VMEM((CB,), jnp.int32),
            pltpu.VMEM((CB, TW), jnp.float32),
            pltpu.VMEM((CB, TW), jnp.float32),
            pltpu.SemaphoreType.DMA,
            pltpu.SemaphoreType.DMA,
        ],
    )
    def k(ta_hbm, tb_hbm, row_hbm, col_hbm, ga_hbm, gb_hbm,
          idr, idc, bufa, bufb, sema, semb):
        cid = lax.axis_index("c")
        sid = lax.axis_index("s")
        wid = sid * 2 + cid

        @pl.loop(0, (NCHUNK + NWORK - 1) // NWORK)
        def _(t):
            ch = wid + t * NWORK

            @pl.when(ch < NCHUNK)
            def _():
                base = pl.multiple_of(ch * CB, 8)
                pltpu.sync_copy(row_hbm.at[pl.ds(base, CB)], idr)
                pltpu.sync_copy(col_hbm.at[pl.ds(base, CB)], idc)
                ca = pltpu.async_copy(ta_hbm.at[idr], bufa, sema)
                cb = pltpu.async_copy(tb_hbm.at[idc], bufb, semb)
                ca.wait()
                cb.wait()
                pltpu.sync_copy(bufa, ga_hbm.at[pl.ds(base, CB)])
                pltpu.sync_copy(bufb, gb_hbm.at[pl.ds(base, CB)])

    return k(ta, tb, row, col)


def _tc_edge_body(ga_ref, gb_ref, ea_ref,
                  wr_ref, c4_ref, eb0_ref, ew1_ref, eb1_ref, ew2_ref, eb2_ref,
                  cw0_ref, cb0_ref, cw1_ref, cb1_ref, cw2_ref, nw0r_ref,
                  out_ref):
    s = ga_ref[...] + gb_ref[...]
    z0p = s[:, :64]
    cdiff = s[:, 64:67]
    radial = jnp.sum(cdiff * cdiff, axis=1, keepdims=True)
    ea = ea_ref[...]
    z0 = z0p + radial * wr_ref[...] + jnp.dot(
        ea, c4_ref[...], preferred_element_type=jnp.float32) + eb0_ref[...]
    z0 = _silu(z0)
    z1 = _silu(_dot_t(z0, ew1_ref[...]) + eb1_ref[...])
    ef = _silu(_dot_t(z1, ew2_ref[...]) + eb2_ref[...])
    c0 = _silu(_dot_t(ef, cw0_ref[...]) + cb0_ref[...])
    c1 = _silu(_dot_t(c0, cw1_ref[...]) + cb1_ref[...])
    cc = jnp.sum(c1 * cw2_ref[...], axis=1, keepdims=True)
    trans = jnp.clip(cdiff * cc, -100.0, 100.0)
    q = _dot_t(ef, nw0r_ref[...])
    nb = s.shape[0]
    ones = jnp.ones((nb, 1), jnp.float32)
    zpad = jnp.zeros((nb, OW - 68), jnp.float32)
    out_ref[...] = jnp.concatenate([q, trans, ones, zpad], axis=1)


def _tc_edge(ga, gb, ea, wr, c4, eb0, ew1, eb1, ew2, eb2,
             cw0, cb0, cw1, cb1, cw2, nw0r):
    blk = 2560
    grid = E // blk
    full = lambda shp: pl.BlockSpec(shp, lambda i: tuple(0 for _ in shp))
    return pl.pallas_call(
        _tc_edge_body,
        grid=(grid,),
        in_specs=[
            pl.BlockSpec((blk, TW), lambda i: (i, 0)),
            pl.BlockSpec((blk, TW), lambda i: (i, 0)),
            pl.BlockSpec((blk, 4), lambda i: (i, 0)),
            full(wr.shape), full(c4.shape), full(eb0.shape),
            full(ew1.shape), full(eb1.shape), full(ew2.shape), full(eb2.shape),
            full(cw0.shape), full(cb0.shape), full(cw1.shape), full(cb1.shape),
            full(cw2.shape), full(nw0r.shape),
        ],
        out_specs=pl.BlockSpec((blk, OW), lambda i: (i, 0)),
        out_shape=jax.ShapeDtypeStruct((E, OW), jnp.float32),
    )(ga, gb, ea, wr, c4, eb0, ew1, eb1, ew2, eb2, cw0, cb0, cw1, cb1, cw2,
      nw0r)


def _sc_scatter(out_e, row):
    mesh = plsc.VectorSubcoreMesh(core_axis_name="c", subcore_axis_name="s")
    halfc = NCHUNK // 2

    @functools.partial(
        pl.kernel,
        out_type=jax.ShapeDtypeStruct((2, N, OW), jnp.float32),
        mesh=mesh,
        scratch_types=[
            pltpu.VMEM((CB,), jnp.int32),
            pltpu.VMEM((CB, OW), jnp.float32),
            pltpu.VMEM((ZB, OW), jnp.float32),
            pltpu.VMEM_SHARED((N, OW), jnp.float32),
        ],
    )
    def k(oe_hbm, row_hbm, p_hbm, idx, buf, zbuf, acc):
        cid = lax.axis_index("c")
        sid = lax.axis_index("s")

        @pl.loop(0, ZB)
        def _(i):
            @pl.loop(0, OW // 16)
            def _(j):
                zbuf[i, pl.ds(j * 16, 16)] = jnp.zeros((16,), jnp.float32)

        @pl.loop(0, (NZB + 15) // 16)
        def _(t):
            blkid = sid + t * 16

            @pl.when(blkid < NZB)
            def _():
                off = pl.multiple_of(blkid * ZB, 8)
                pltpu.sync_copy(zbuf, acc.at[pl.ds(off, ZB)])

        plsc.subcore_barrier()

        @pl.loop(0, (halfc + 15) // 16)
        def _(t):
            kchunk = sid + t * 16

            @pl.when(kchunk < halfc)
            def _():
                base = pl.multiple_of((cid * halfc + kchunk) * CB, 8)
                pltpu.sync_copy(row_hbm.at[pl.ds(base, CB)], idx)
                pltpu.sync_copy(oe_hbm.at[pl.ds(base, CB)], buf)
                pltpu.sync_copy(buf, acc.at[idx], add=True)

        plsc.subcore_barrier()

        @pl.loop(0, (NZB + 15) // 16)
        def _(t):
            blkid = sid + t * 16

            @pl.when(blkid < NZB)
            def _():
                off = pl.multiple_of(blkid * ZB, 8)
                pltpu.sync_copy(acc.at[pl.ds(off, ZB)], zbuf)
                pltpu.sync_copy(zbuf, p_hbm.at[cid, pl.ds(off, ZB)])

    return k(out_e, row)


def _tc_post_body(h_ref, c_ref, p_ref,
                  nw0l_ref, nb0_ref, nw1_ref, nb1_ref, nw2_ref, nb2_ref,
                  hout_ref, cout_ref):
    acc = p_ref[0] + p_ref[1]
    qagg = acc[:, :64]
    ts = acc[:, 64:67]
    cnt = acc[:, 67:68]
    h = h_ref[...]
    y = _silu(_dot_t(h, nw0l_ref[...]) + qagg + nb0_ref[...])
    y = _silu(_dot_t(y, nw1_ref[...]) + nb1_ref[...])
    y = _dot_t(y, nw2_ref[...]) + nb2_ref[...]
    hout_ref[...] = h + y
    cout_ref[...] = c_ref[...] + ts / jnp.clip(cnt, 1.0, None)


def _tc_post(h, coord, p, nw0l, nb0, nw1, nb1, nw2, nb2):
    blk = 2000
    grid = N // blk
    full = lambda shp: pl.BlockSpec(shp, lambda i: tuple(0 for _ in shp))
    return pl.pallas_call(
        _tc_post_body,
        grid=(grid,),
        in_specs=[
            pl.BlockSpec((blk, D), lambda i: (i, 0)),
            pl.BlockSpec((blk, 3), lambda i: (i, 0)),
            pl.BlockSpec((2, blk, OW), lambda i: (0, i, 0)),
            full(nw0l.shape), full(nb0.shape), full(nw1.shape),
            full(nb1.shape), full(nw2.shape), full(nb2.shape),
        ],
        out_specs=[
            pl.BlockSpec((blk, D), lambda i: (i, 0)),
            pl.BlockSpec((blk, 3), lambda i: (i, 0)),
        ],
        out_shape=[
            jax.ShapeDtypeStruct((N, D), jnp.float32),
            jax.ShapeDtypeStruct((N, 3), jnp.float32),
        ],
    )(h, coord, p, nw0l, nb0, nw1, nb1, nw2, nb2)


def kernel(h, edge_index, coord, edge_attr,
           ew0, eb0, ew1, eb1, ew2, eb2,
           nw0, nb0, nw1, nb1, nw2, nb2,
           cw0, cb0, cw1, cb1, cw2):
    row = edge_index[0]
    col = edge_index[1]
    a = ew0[:, :D]
    b = ew0[:, D:2 * D]
    wr = ew0[:, 2 * D:2 * D + 1].T
    c4 = ew0[:, 2 * D + 1:].T

    nw0l = nw0[:, :D]
    nw0r = nw0[:, D:]

    ta, tb = _tc_pre(h, coord, a, b)
    ga, gb = _sc_gather(ta, tb, row, col)
    out_e = _tc_edge(ga, gb, edge_attr, wr, c4,
                     eb0.reshape(1, -1), ew1, eb1.reshape(1, -1),
                     ew2, eb2.reshape(1, -1),
                     cw0, cb0.reshape(1, -1), cw1, cb1.reshape(1, -1), cw2,
                     nw0r)
    p = _sc_scatter(out_e, row)
    h_out, coord_out = _tc_post(h, coord, p, nw0l, nb0.reshape(1, -1),
                                nw1, nb1.reshape(1, -1), nw2, nb2.reshape(1, -1))
    return (h_out, coord_out, edge_attr)


# speedup vs baseline: 5.4237x; 1.1933x over previous
import functools

import jax
import jax.numpy as jnp
from jax import lax
from jax.experimental import pallas as pl
from jax.experimental.pallas import tpu as pltpu
from jax.experimental.pallas import tpu_sc as plsc

N = 10000
E = 320000
D = 128
TW = 128
OW = 128
CB = 128
NCHUNK = E // CB
NWORK = 32
ZB = 200
NZB = N // ZB


def _silu(x):
    return x * jax.nn.sigmoid(x)


def _dot_t(x, w):
    return lax.dot_general(x, w, (((1,), (1,)), ((), ())),
                           preferred_element_type=jnp.float32)


def _dot_t_bf(x, w):
    return lax.dot_general(x.astype(jnp.bfloat16), w.astype(jnp.bfloat16),
                           (((1,), (1,)), ((), ())),
                           preferred_element_type=jnp.float32)


def _tc_pre_body(h_ref, c_ref, a_ref, b_ref, ta_ref, tb_ref):
    h = h_ref[...]
    cd = c_ref[...]
    pad = jnp.zeros((h.shape[0], TW - 67), jnp.float32)
    ha = _dot_t(h, a_ref[...])
    hb = _dot_t(h, b_ref[...])
    ta_ref[...] = jnp.concatenate([ha, cd, pad], axis=1)
    tb_ref[...] = jnp.concatenate([hb, -cd, pad], axis=1)


def _tc_pre(h, coord, a, b):
    blk = 2000
    grid = N // blk
    return pl.pallas_call(
        _tc_pre_body,
        grid=(grid,),
        in_specs=[
            pl.BlockSpec((blk, D), lambda i: (i, 0)),
            pl.BlockSpec((blk, 3), lambda i: (i, 0)),
            pl.BlockSpec((64, D), lambda i: (0, 0)),
            pl.BlockSpec((64, D), lambda i: (0, 0)),
        ],
        out_specs=[
            pl.BlockSpec((blk, TW), lambda i: (i, 0)),
            pl.BlockSpec((blk, TW), lambda i: (i, 0)),
        ],
        out_shape=[
            jax.ShapeDtypeStruct((N, TW), jnp.float32),
            jax.ShapeDtypeStruct((N, TW), jnp.float32),
        ],
    )(h, coord, a, b)


def _sc_gather(ta, tb, row, col):
    mesh = plsc.VectorSubcoreMesh(core_axis_name="c", subcore_axis_name="s")

    @functools.partial(
        pl.kernel,
        out_type=[
            jax.ShapeDtypeStruct((E, TW), jnp.float32),
            jax.ShapeDtypeStruct((E, TW), jnp.float32),
        ],
        mesh=mesh,
        scratch_types=[
            pltpu.VMEM((CB,), jnp.int32),
            pltpu.VMEM((CB,), jnp.int32),
            pltpu.VMEM((CB, TW), jnp.float32),
            pltpu.VMEM((CB, TW), jnp.float32),
            pltpu.SemaphoreType.DMA,
            pltpu.SemaphoreType.DMA,
        ],
    )
    def k(ta_hbm, tb_hbm, row_hbm, col_hbm, ga_hbm, gb_hbm,
          idr, idc, bufa, bufb, sema, semb):
        cid = lax.axis_index("c")
        sid = lax.axis_index("s")
        wid = sid * 2 + cid

        @pl.loop(0, (NCHUNK + NWORK - 1) // NWORK)
        def _(t):
            ch = wid + t * NWORK

            @pl.when(ch < NCHUNK)
            def _():
                base = pl.multiple_of(ch * CB, 8)
                pltpu.sync_copy(row_hbm.at[pl.ds(base, CB)], idr)
                pltpu.sync_copy(col_hbm.at[pl.ds(base, CB)], idc)
                ca = pltpu.async_copy(ta_hbm.at[idr], bufa, sema)
                cb = pltpu.async_copy(tb_hbm.at[idc], bufb, semb)
                ca.wait()
                cb.wait()
                pltpu.sync_copy(bufa, ga_hbm.at[pl.ds(base, CB)])
                pltpu.sync_copy(bufb, gb_hbm.at[pl.ds(base, CB)])

    return k(ta, tb, row, col)


def _tc_edge_body(ga_ref, gb_ref, ea_ref,
                  wr_ref, c4_ref, eb0_ref, ew1_ref, eb1_ref, ew2_ref, eb2_ref,
                  cw0_ref, cb0_ref, cw1_ref, cb1_ref, cw2_ref, nw0r_ref,
                  out_ref):
    s = ga_ref[...] + gb_ref[...]
    z0p = s[:, :64]
    cdiff = s[:, 64:67]
    radial = jnp.sum(cdiff * cdiff, axis=1, keepdims=True)
    ea = ea_ref[...]
    z0 = z0p + radial * wr_ref[...] + jnp.dot(
        ea, c4_ref[...], preferred_element_type=jnp.float32) + eb0_ref[...]
    z0 = _silu(z0)
    z1 = _silu(_dot_t_bf(z0, ew1_ref[...]) + eb1_ref[...])
    ef = _silu(_dot_t_bf(z1, ew2_ref[...]) + eb2_ref[...])
    c0 = _silu(_dot_t_bf(ef, cw0_ref[...]) + cb0_ref[...])
    c1 = _silu(_dot_t_bf(c0, cw1_ref[...]) + cb1_ref[...])
    cc = jnp.sum(c1 * cw2_ref[...], axis=1, keepdims=True)
    trans = jnp.clip(cdiff * cc, -100.0, 100.0)
    q = _dot_t_bf(ef, nw0r_ref[...])
    nb = s.shape[0]
    ones = jnp.ones((nb, 1), jnp.float32)
    zpad = jnp.zeros((nb, OW - 68), jnp.float32)
    out_ref[...] = jnp.concatenate([q, trans, ones, zpad], axis=1)


def _tc_edge(ga, gb, ea, wr, c4, eb0, ew1, eb1, ew2, eb2,
             cw0, cb0, cw1, cb1, cw2, nw0r):
    blk = 2560
    grid = E // blk
    full = lambda shp: pl.BlockSpec(shp, lambda i: tuple(0 for _ in shp))
    return pl.pallas_call(
        _tc_edge_body,
        grid=(grid,),
        in_specs=[
            pl.BlockSpec((blk, TW), lambda i: (i, 0)),
            pl.BlockSpec((blk, TW), lambda i: (i, 0)),
            pl.BlockSpec((blk, 4), lambda i: (i, 0)),
            full(wr.shape), full(c4.shape), full(eb0.shape),
            full(ew1.shape), full(eb1.shape), full(ew2.shape), full(eb2.shape),
            full(cw0.shape), full(cb0.shape), full(cw1.shape), full(cb1.shape),
            full(cw2.shape), full(nw0r.shape),
        ],
        out_specs=pl.BlockSpec((blk, OW), lambda i: (i, 0)),
        out_shape=jax.ShapeDtypeStruct((E, OW), jnp.float32),
    )(ga, gb, ea, wr, c4, eb0, ew1, eb1, ew2, eb2, cw0, cb0, cw1, cb1, cw2,
      nw0r)


def _sc_scatter(out_e, row):
    mesh = plsc.VectorSubcoreMesh(core_axis_name="c", subcore_axis_name="s")
    halfc = NCHUNK // 2

    @functools.partial(
        pl.kernel,
        out_type=jax.ShapeDtypeStruct((2, N, OW), jnp.float32),
        mesh=mesh,
        scratch_types=[
            pltpu.VMEM((CB,), jnp.int32),
            pltpu.VMEM((CB, OW), jnp.float32),
            pltpu.VMEM((ZB, OW), jnp.float32),
            pltpu.VMEM_SHARED((N, OW), jnp.float32),
        ],
    )
    def k(oe_hbm, row_hbm, p_hbm, idx, buf, zbuf, acc):
        cid = lax.axis_index("c")
        sid = lax.axis_index("s")

        @pl.loop(0, ZB)
        def _(i):
            @pl.loop(0, OW // 16)
            def _(j):
                zbuf[i, pl.ds(j * 16, 16)] = jnp.zeros((16,), jnp.float32)

        @pl.loop(0, (NZB + 15) // 16)
        def _(t):
            blkid = sid + t * 16

            @pl.when(blkid < NZB)
            def _():
                off = pl.multiple_of(blkid * ZB, 8)
                pltpu.sync_copy(zbuf, acc.at[pl.ds(off, ZB)])

        plsc.subcore_barrier()

        @pl.loop(0, (halfc + 15) // 16)
        def _(t):
            kchunk = sid + t * 16

            @pl.when(kchunk < halfc)
            def _():
                base = pl.multiple_of((cid * halfc + kchunk) * CB, 8)
                pltpu.sync_copy(row_hbm.at[pl.ds(base, CB)], idx)
                pltpu.sync_copy(oe_hbm.at[pl.ds(base, CB)], buf)
                pltpu.sync_copy(buf, acc.at[idx], add=True)

        plsc.subcore_barrier()

        @pl.loop(0, (NZB + 15) // 16)
        def _(t):
            blkid = sid + t * 16

            @pl.when(blkid < NZB)
            def _():
                off = pl.multiple_of(blkid * ZB, 8)
                pltpu.sync_copy(acc.at[pl.ds(off, ZB)], zbuf)
                pltpu.sync_copy(zbuf, p_hbm.at[cid, pl.ds(off, ZB)])

    return k(out_e, row)


def _tc_post_body(h_ref, c_ref, p_ref,
                  nw0l_ref, nb0_ref, nw1_ref, nb1_ref, nw2_ref, nb2_ref,
                  hout_ref, cout_ref):
    acc = p_ref[0] + p_ref[1]
    qagg = acc[:, :64]
    ts = acc[:, 64:67]
    cnt = acc[:, 67:68]
    h = h_ref[...]
    y = _silu(_dot_t(h, nw0l_ref[...]) + qagg + nb0_ref[...])
    y = _silu(_dot_t(y, nw1_ref[...]) + nb1_ref[...])
    y = _dot_t(y, nw2_ref[...]) + nb2_ref[...]
    hout_ref[...] = h + y
    cout_ref[...] = c_ref[...] + ts / jnp.clip(cnt, 1.0, None)


def _tc_post(h, coord, p, nw0l, nb0, nw1, nb1, nw2, nb2):
    blk = 2000
    grid = N // blk
    full = lambda shp: pl.BlockSpec(shp, lambda i: tuple(0 for _ in shp))
    return pl.pallas_call(
        _tc_post_body,
        grid=(grid,),
        in_specs=[
            pl.BlockSpec((blk, D), lambda i: (i, 0)),
            pl.BlockSpec((blk, 3), lambda i: (i, 0)),
            pl.BlockSpec((2, blk, OW), lambda i: (0, i, 0)),
            full(nw0l.shape), full(nb0.shape), full(nw1.shape),
            full(nb1.shape), full(nw2.shape), full(nb2.shape),
        ],
        out_specs=[
            pl.BlockSpec((blk, D), lambda i: (i, 0)),
            pl.BlockSpec((blk, 3), lambda i: (i, 0)),
        ],
        out_shape=[
            jax.ShapeDtypeStruct((N, D), jnp.float32),
            jax.ShapeDtypeStruct((N, 3), jnp.float32),
        ],
    )(h, coord, p, nw0l, nb0, nw1, nb1, nw2, nb2)


def kernel(h, edge_index, coord, edge_attr,
           ew0, eb0, ew1, eb1, ew2, eb2,
           nw0, nb0, nw1, nb1, nw2, nb2,
           cw0, cb0, cw1, cb1, cw2):
    row = edge_index[0]
    col = edge_index[1]
    a = ew0[:, :D]
    b = ew0[:, D:2 * D]
    wr = ew0[:, 2 * D:2 * D + 1].T
    c4 = ew0[:, 2 * D + 1:].T

    nw0l = nw0[:, :D]
    nw0r = nw0[:, D:]

    ta, tb = _tc_pre(h, coord, a, b)
    ga, gb = _sc_gather(ta, tb, row, col)
    out_e = _tc_edge(ga, gb, edge_attr, wr, c4,
                     eb0.reshape(1, -1), ew1, eb1.reshape(1, -1),
                     ew2, eb2.reshape(1, -1),
                     cw0, cb0.reshape(1, -1), cw1, cb1.reshape(1, -1), cw2,
                     nw0r)
    p = _sc_scatter(out_e, row)
    h_out, coord_out = _tc_post(h, coord, p, nw0l, nb0.reshape(1, -1),
                                nw1, nb1.reshape(1, -1), nw2, nb2.reshape(1, -1))
    return (h_out, coord_out, edge_attr)


# speedup vs baseline: 7.2420x; 1.3352x over previous
import functools

import jax
import jax.numpy as jnp
from jax import lax
from jax.experimental import pallas as pl
from jax.experimental.pallas import tpu as pltpu
from jax.experimental.pallas import tpu_sc as plsc

N = 10000
E = 320000
D = 128
TW = 128
OW = 128
CB = 128
NWORK = 32
ZB = 200
NZB = N // ZB
NSLAB = 4
SL = E // NSLAB


def _silu(x):
    return x * jax.nn.sigmoid(x)


def _dot_t(x, w):
    return lax.dot_general(x, w, (((1,), (1,)), ((), ())),
                           preferred_element_type=jnp.float32)


def _dot_t_bf(x, w):
    return lax.dot_general(x.astype(jnp.bfloat16), w.astype(jnp.bfloat16),
                           (((1,), (1,)), ((), ())),
                           preferred_element_type=jnp.float32)


def _tc_pre_body(h_ref, c_ref, a_ref, b_ref, ta_ref, tb_ref):
    h = h_ref[...]
    cd = c_ref[...]
    pad = jnp.zeros((h.shape[0], TW - 67), jnp.float32)
    ha = _dot_t(h, a_ref[...])
    hb = _dot_t(h, b_ref[...])
    ta_ref[...] = jnp.concatenate([ha, cd, pad], axis=1)
    tb_ref[...] = jnp.concatenate([hb, -cd, pad], axis=1)


def _tc_pre(h, coord, a, b):
    blk = 2000
    grid = N // blk
    return pl.pallas_call(
        _tc_pre_body,
        grid=(grid,),
        in_specs=[
            pl.BlockSpec((blk, D), lambda i: (i, 0)),
            pl.BlockSpec((blk, 3), lambda i: (i, 0)),
            pl.BlockSpec((64, D), lambda i: (0, 0)),
            pl.BlockSpec((64, D), lambda i: (0, 0)),
        ],
        out_specs=[
            pl.BlockSpec((blk, TW), lambda i: (i, 0)),
            pl.BlockSpec((blk, TW), lambda i: (i, 0)),
        ],
        out_shape=[
            jax.ShapeDtypeStruct((N, TW), jnp.float32),
            jax.ShapeDtypeStruct((N, TW), jnp.float32),
        ],
    )(h, coord, a, b)


def _sc_gather(ta, tb, row, col):
    mesh = plsc.VectorSubcoreMesh(core_axis_name="c", subcore_axis_name="s")

    ne = row.shape[0]
    nchunk = ne // CB

    @functools.partial(
        pl.kernel,
        out_type=[
            jax.ShapeDtypeStruct((ne, TW), jnp.float32),
            jax.ShapeDtypeStruct((ne, TW), jnp.float32),
        ],
        mesh=mesh,
        scratch_types=[
            pltpu.VMEM((CB,), jnp.int32),
            pltpu.VMEM((CB,), jnp.int32),
            pltpu.VMEM((CB, TW), jnp.float32),
            pltpu.VMEM((CB, TW), jnp.float32),
            pltpu.SemaphoreType.DMA,
            pltpu.SemaphoreType.DMA,
        ],
    )
    def k(ta_hbm, tb_hbm, row_hbm, col_hbm, ga_hbm, gb_hbm,
          idr, idc, bufa, bufb, sema, semb):
        cid = lax.axis_index("c")
        sid = lax.axis_index("s")
        wid = sid * 2 + cid

        @pl.loop(0, (nchunk + NWORK - 1) // NWORK)
        def _(t):
            ch = wid + t * NWORK

            @pl.when(ch < nchunk)
            def _():
                base = pl.multiple_of(ch * CB, 8)
                pltpu.sync_copy(row_hbm.at[pl.ds(base, CB)], idr)
                pltpu.sync_copy(col_hbm.at[pl.ds(base, CB)], idc)
                ca = pltpu.async_copy(ta_hbm.at[idr], bufa, sema)
                cb = pltpu.async_copy(tb_hbm.at[idc], bufb, semb)
                ca.wait()
                cb.wait()
                pltpu.sync_copy(bufa, ga_hbm.at[pl.ds(base, CB)])
                pltpu.sync_copy(bufb, gb_hbm.at[pl.ds(base, CB)])

    return k(ta, tb, row, col)


def _tc_edge_body(ga_ref, gb_ref, ea_ref,
                  wr_ref, c4_ref, eb0_ref, ew1_ref, eb1_ref, ew2_ref, eb2_ref,
                  cw0_ref, cb0_ref, cw1_ref, cb1_ref, cw2_ref, nw0r_ref,
                  out_ref):
    s = ga_ref[...] + gb_ref[...]
    z0p = s[:, :64]
    cdiff = s[:, 64:67]
    radial = jnp.sum(cdiff * cdiff, axis=1, keepdims=True)
    ea = ea_ref[...]
    z0 = z0p + radial * wr_ref[...] + jnp.dot(
        ea, c4_ref[...], preferred_element_type=jnp.float32) + eb0_ref[...]
    z0 = _silu(z0)
    z1 = _silu(_dot_t_bf(z0, ew1_ref[...]) + eb1_ref[...])
    ef = _silu(_dot_t_bf(z1, ew2_ref[...]) + eb2_ref[...])
    c0 = _silu(_dot_t_bf(ef, cw0_ref[...]) + cb0_ref[...])
    c1 = _silu(_dot_t_bf(c0, cw1_ref[...]) + cb1_ref[...])
    cc = jnp.sum(c1 * cw2_ref[...], axis=1, keepdims=True)
    trans = jnp.clip(cdiff * cc, -100.0, 100.0)
    q = _dot_t_bf(ef, nw0r_ref[...])
    nb = s.shape[0]
    ones = jnp.ones((nb, 1), jnp.float32)
    zpad = jnp.zeros((nb, OW - 68), jnp.float32)
    out_ref[...] = jnp.concatenate([q, trans, ones, zpad], axis=1)


def _tc_edge(ga, gb, ea, wr, c4, eb0, ew1, eb1, ew2, eb2,
             cw0, cb0, cw1, cb1, cw2, nw0r):
    blk = 3200
    ne = ga.shape[0]
    grid = ne // blk
    full = lambda shp: pl.BlockSpec(shp, lambda i: tuple(0 for _ in shp))
    return pl.pallas_call(
        _tc_edge_body,
        grid=(grid,),
        in_specs=[
            pl.BlockSpec((blk, TW), lambda i: (i, 0)),
            pl.BlockSpec((blk, TW), lambda i: (i, 0)),
            pl.BlockSpec((blk, 4), lambda i: (i, 0)),
            full(wr.shape), full(c4.shape), full(eb0.shape),
            full(ew1.shape), full(eb1.shape), full(ew2.shape), full(eb2.shape),
            full(cw0.shape), full(cb0.shape), full(cw1.shape), full(cb1.shape),
            full(cw2.shape), full(nw0r.shape),
        ],
        out_specs=pl.BlockSpec((blk, OW), lambda i: (i, 0)),
        out_shape=jax.ShapeDtypeStruct((ne, OW), jnp.float32),
    )(ga, gb, ea, wr, c4, eb0, ew1, eb1, ew2, eb2, cw0, cb0, cw1, cb1, cw2,
      nw0r)


def _sc_scatter(out_e, row):
    mesh = plsc.VectorSubcoreMesh(core_axis_name="c", subcore_axis_name="s")
    nchunk = row.shape[0] // CB
    maxk = (nchunk + 1) // 2

    @functools.partial(
        pl.kernel,
        out_type=jax.ShapeDtypeStruct((2, N, OW), jnp.float32),
        mesh=mesh,
        scratch_types=[
            pltpu.VMEM((CB,), jnp.int32),
            pltpu.VMEM((CB, OW), jnp.float32),
            pltpu.VMEM((ZB, OW), jnp.float32),
            pltpu.VMEM_SHARED((N, OW), jnp.float32),
        ],
    )
    def k(oe_hbm, row_hbm, p_hbm, idx, buf, zbuf, acc):
        cid = lax.axis_index("c")
        sid = lax.axis_index("s")

        @pl.loop(0, ZB)
        def _(i):
            @pl.loop(0, OW // 16)
            def _(j):
                zbuf[i, pl.ds(j * 16, 16)] = jnp.zeros((16,), jnp.float32)

        @pl.loop(0, (NZB + 15) // 16)
        def _(t):
            blkid = sid + t * 16

            @pl.when(blkid < NZB)
            def _():
                off = pl.multiple_of(blkid * ZB, 8)
                pltpu.sync_copy(zbuf, acc.at[pl.ds(off, ZB)])

        plsc.subcore_barrier()

        @pl.loop(0, (maxk + 15) // 16)
        def _(t):
            kchunk = sid + t * 16
            ch = kchunk * 2 + cid

            @pl.when(ch < nchunk)
            def _():
                base = pl.multiple_of(ch * CB, 8)
                pltpu.sync_copy(row_hbm.at[pl.ds(base, CB)], idx)
                pltpu.sync_copy(oe_hbm.at[pl.ds(base, CB)], buf)
                pltpu.sync_copy(buf, acc.at[idx], add=True)

        plsc.subcore_barrier()

        @pl.loop(0, (NZB + 15) // 16)
        def _(t):
            blkid = sid + t * 16

            @pl.when(blkid < NZB)
            def _():
                off = pl.multiple_of(blkid * ZB, 8)
                pltpu.sync_copy(acc.at[pl.ds(off, ZB)], zbuf)
                pltpu.sync_copy(zbuf, p_hbm.at[cid, pl.ds(off, ZB)])

    return k(out_e, row)


def _tc_post_body(h_ref, c_ref, *rest):
    p_refs = rest[:NSLAB]
    (nw0l_ref, nb0_ref, nw1_ref, nb1_ref, nw2_ref, nb2_ref,
     hout_ref, cout_ref) = rest[NSLAB:]
    acc = p_refs[0][0] + p_refs[0][1]
    for pr in p_refs[1:]:
        acc = acc + pr[0] + pr[1]
    qagg = acc[:, :64]
    ts = acc[:, 64:67]
    cnt = acc[:, 67:68]
    h = h_ref[...]
    y = _silu(_dot_t(h, nw0l_ref[...]) + qagg + nb0_ref[...])
    y = _silu(_dot_t(y, nw1_ref[...]) + nb1_ref[...])
    y = _dot_t(y, nw2_ref[...]) + nb2_ref[...]
    hout_ref[...] = h + y
    cout_ref[...] = c_ref[...] + ts / jnp.clip(cnt, 1.0, None)


def _tc_post(h, coord, ps, nw0l, nb0, nw1, nb1, nw2, nb2):
    blk = 2000
    grid = N // blk
    full = lambda shp: pl.BlockSpec(shp, lambda i: tuple(0 for _ in shp))
    return pl.pallas_call(
        _tc_post_body,
        grid=(grid,),
        in_specs=[
            pl.BlockSpec((blk, D), lambda i: (i, 0)),
            pl.BlockSpec((blk, 3), lambda i: (i, 0)),
        ] + [
            pl.BlockSpec((2, blk, OW), lambda i: (0, i, 0)) for _ in ps
        ] + [
            full(nw0l.shape), full(nb0.shape), full(nw1.shape),
            full(nb1.shape), full(nw2.shape), full(nb2.shape),
        ],
        out_specs=[
            pl.BlockSpec((blk, D), lambda i: (i, 0)),
            pl.BlockSpec((blk, 3), lambda i: (i, 0)),
        ],
        out_shape=[
            jax.ShapeDtypeStruct((N, D), jnp.float32),
            jax.ShapeDtypeStruct((N, 3), jnp.float32),
        ],
    )(h, coord, *ps, nw0l, nb0, nw1, nb1, nw2, nb2)


def kernel(h, edge_index, coord, edge_attr,
           ew0, eb0, ew1, eb1, ew2, eb2,
           nw0, nb0, nw1, nb1, nw2, nb2,
           cw0, cb0, cw1, cb1, cw2):
    row = edge_index[0]
    col = edge_index[1]
    a = ew0[:, :D]
    b = ew0[:, D:2 * D]
    wr = ew0[:, 2 * D:2 * D + 1].T
    c4 = ew0[:, 2 * D + 1:].T

    nw0l = nw0[:, :D]
    nw0r = nw0[:, D:]

    ta, tb = _tc_pre(h, coord, a, b)
    ps = []
    for si in range(NSLAB):
        sl = slice(si * SL, (si + 1) * SL)
        row_s = row[sl]
        ga, gb = _sc_gather(ta, tb, row_s, col[sl])
        out_e = _tc_edge(ga, gb, edge_attr[sl], wr, c4,
                         eb0.reshape(1, -1), ew1, eb1.reshape(1, -1),
                         ew2, eb2.reshape(1, -1),
                         cw0, cb0.reshape(1, -1), cw1, cb1.reshape(1, -1), cw2,
                         nw0r)
        ps.append(_sc_scatter(out_e, row_s))
    h_out, coord_out = _tc_post(h, coord, ps, nw0l, nb0.reshape(1, -1),
                                nw1, nb1.reshape(1, -1), nw2, nb2.reshape(1, -1))
    return (h_out, coord_out, edge_attr)
